# Initial kernel scaffold; baseline (speedup 1.0000x reference)
import jax
import jax.numpy as jnp
from jax.experimental import pallas as pl


def kernel(x, edge_index, ggnn_w, gru_w_ih, gru_w_hh, gru_b_ih, gru_b_hh, cw1_w, cw1_b, cw2_w, cw2_b, cw3_w, cw3_b, cn1_w, cn1_b, cn2_w, cn2_b, cn3_w, cn3_b, mw_w, mw_b, mn_w, mn_b):
    raise NotImplementedError("write your pallas kernel here")


# speedup vs baseline: 4.1453x; 4.1453x over previous
import functools

import numpy as np

import jax
import jax.numpy as jnp
from jax import lax
from jax.experimental import pallas as pl
from jax.experimental.pallas import tpu as pltpu
from jax.experimental.pallas import tpu_sc as plsc

_N = 10000
_E = 320000
_H = 128
_DIN = 128
_L = 4
_NPG = 200
_B = _N // _NPG

_NTILES = 32
_CH = 128
_NCH = 79
_PT = _CH * _NCH
_EPAD = _NTILES * _PT
_NPAD = 10240
_RPT = _NPAD // 16


def _sc_scatter_body(m_hbm, zeros_hbm, src_hbm, dst_hbm, out_hbm,
                     src_v, dst_v, rows_v, acc_sh, sem):
    cid = lax.axis_index("c")
    sid = lax.axis_index("s")
    wid = cid * 16 + sid
    r0 = sid * _RPT
    pltpu.sync_copy(zeros_hbm.at[pl.ds(r0, _RPT)], acc_sh.at[pl.ds(r0, _RPT)])
    pltpu.sync_copy(src_hbm.at[wid], src_v)
    pltpu.sync_copy(dst_hbm.at[wid], dst_v)
    plsc.subcore_barrier()

    def body(j):
        pltpu.async_copy(m_hbm.at[src_v.at[j]], rows_v, sem).wait()
        pltpu.sync_copy(rows_v, acc_sh.at[dst_v.at[j]], add=True)

    pl.loop(0, _NCH)(body)
    plsc.subcore_barrier()
    for k in range(_RPT // _CH):
        rr = r0 + k * _CH
        pltpu.sync_copy(acc_sh.at[pl.ds(rr, _CH)], rows_v)
        pltpu.sync_copy(rows_v, out_hbm.at[cid, pl.ds(rr, _CH)])


@functools.cache
def _get_sc_scatter():
    return pl.kernel(
        _sc_scatter_body,
        out_type=jax.ShapeDtypeStruct((2, _NPAD, _H), jnp.float32),
        mesh=plsc.VectorSubcoreMesh(core_axis_name="c", subcore_axis_name="s"),
        scratch_types=[
            pltpu.VMEM((_NCH, _CH), jnp.int32),
            pltpu.VMEM((_NCH, _CH), jnp.int32),
            pltpu.VMEM((_CH, _H), jnp.float32),
            pltpu.VMEM_SHARED((_NPAD, _H), jnp.float32),
            pltpu.SemaphoreType.DMA,
        ],
    )


_NB = 1000
_NG = _N // _NB


def _mm_body(x_ref, w_ref, o_ref):
    o_ref[...] = jnp.dot(x_ref[...], w_ref[...],
                         preferred_element_type=jnp.float32)


_m0 = pl.pallas_call(
    _mm_body,
    grid=(_NG,),
    in_specs=[
        pl.BlockSpec((_NB, _H), lambda i: (i, 0)),
        pl.BlockSpec((_H, _H), lambda i: (0, 0)),
    ],
    out_specs=pl.BlockSpec((_NB, _H), lambda i: (i, 0)),
    out_shape=jax.ShapeDtypeStruct((_N, _H), jnp.float32),
)


def _gru_body(h_ref, agg_ref, wih_ref, whh_ref, bih_ref, bhh_ref, wnext_ref,
              hout_ref, mout_ref):
    h = h_ref[...]
    agg = agg_ref[0] + agg_ref[1]
    gi = jnp.dot(agg, wih_ref[...], preferred_element_type=jnp.float32)
    gi = gi + bih_ref[...]
    gh = jnp.dot(h, whh_ref[...], preferred_element_type=jnp.float32)
    gh = gh + bhh_ref[...]
    r = jax.nn.sigmoid(gi[:, :_H] + gh[:, :_H])
    z = jax.nn.sigmoid(gi[:, _H:2 * _H] + gh[:, _H:2 * _H])
    n = jnp.tanh(gi[:, 2 * _H:] + r * gh[:, 2 * _H:])
    hn = (1.0 - z) * n + z * h
    hout_ref[...] = hn
    mout_ref[...] = jnp.dot(hn, wnext_ref[...],
                            preferred_element_type=jnp.float32)


_gru = pl.pallas_call(
    _gru_body,
    grid=(_NG,),
    in_specs=[
        pl.BlockSpec((_NB, _H), lambda i: (i, 0)),
        pl.BlockSpec((2, _NB, _H), lambda i: (0, i, 0)),
        pl.BlockSpec((_H, 3 * _H), lambda i: (0, 0)),
        pl.BlockSpec((_H, 3 * _H), lambda i: (0, 0)),
        pl.BlockSpec((1, 3 * _H), lambda i: (0, 0)),
        pl.BlockSpec((1, 3 * _H), lambda i: (0, 0)),
        pl.BlockSpec((_H, _H), lambda i: (0, 0)),
    ],
    out_specs=[
        pl.BlockSpec((_NB, _H), lambda i: (i, 0)),
        pl.BlockSpec((_NB, _H), lambda i: (i, 0)),
    ],
    out_shape=[
        jax.ShapeDtypeStruct((_N, _H), jnp.float32),
        jax.ShapeDtypeStruct((_N, _H), jnp.float32),
    ],
)


def _pool_mats(t_in, k, s):
    t_out = (t_in - k) // s + 1
    mats = np.zeros((k, t_out, t_in), np.float32)
    for off in range(k):
        for t in range(t_out):
            mats[off, t, s * t + off] = 1.0
    return mats


def _conv_path_2d(z, w1, b1, s1, w2, b2, s2, w3, b3, u):
    t1 = z.shape[0] - 2
    acc = None
    for k in range(3):
        c = jnp.dot(z[k:k + t1, :], w1[k],
                    preferred_element_type=jnp.float32)
        acc = c if acc is None else acc + c
    y1 = jnp.maximum(acc + b1, 0.0)
    p = None
    for off in range(s1.shape[0]):
        sel = jnp.dot(s1[off], y1, preferred_element_type=jnp.float32)
        p = sel if p is None else jnp.maximum(p, sel)
    y2 = jnp.maximum(jnp.dot(p, w2, preferred_element_type=jnp.float32)
                     + b2, 0.0)
    q = None
    for off in range(s2.shape[0]):
        sel = jnp.dot(s2[off], y2, preferred_element_type=jnp.float32)
        q = sel if q is None else jnp.maximum(q, sel)
    t4 = q.shape[0] - 2
    acc3 = None
    for k in range(3):
        c = jnp.dot(q[k:k + t4, :], w3[k],
                    preferred_element_type=jnp.float32)
        acc3 = c if acc3 is None else acc3 + c
    y3 = jnp.maximum(acc3 + b3, 0.0)
    return jnp.dot(u, y3, preferred_element_type=jnp.float32)


def _head_body(zw_ref, zn_ref,
               w1w_ref, b1w_ref, s1w_ref, w2w_ref, b2w_ref, s2w_ref,
               w3w_ref, b3w_ref, uw_ref, mw_ref, mwb_ref,
               w1n_ref, b1n_ref, s1n_ref, w2n_ref, b2n_ref, s2n_ref,
               w3n_ref, b3n_ref, un_ref, mn_ref, mnb_ref,
               out_ref):
    yw = _conv_path_2d(zw_ref[0], w1w_ref[...], b1w_ref[...], s1w_ref[...],
                       w2w_ref[...], b2w_ref[...], s2w_ref[...],
                       w3w_ref[...], b3w_ref[...], uw_ref[...])
    yn = _conv_path_2d(zn_ref[0], w1n_ref[...], b1n_ref[...], s1n_ref[...],
                       w2n_ref[...], b2n_ref[...], s2n_ref[...],
                       w3n_ref[...], b3n_ref[...], un_ref[...])
    ywh = jnp.dot(mw_ref[...], yw, preferred_element_type=jnp.float32)
    ywh = ywh + mwb_ref[...]
    ynh = jnp.dot(mn_ref[...], yn, preferred_element_type=jnp.float32)
    ynh = ynh + mnb_ref[...]
    sig = jax.nn.sigmoid(ywh * ynh)
    out_ref[...] = jnp.broadcast_to(sig[None], (1, 1, 128))


def _full(shape):
    return pl.BlockSpec(shape, lambda b: tuple(0 for _ in shape))


_head = pl.pallas_call(
    _head_body,
    grid=(_B,),
    in_specs=[
        pl.BlockSpec((1, _H + _DIN, _NPG), lambda b: (b, 0, 0)),
        pl.BlockSpec((1, _H, _NPG), lambda b: (b, 0, 0)),
        _full((3, _NPG, 100)), _full((1, 100)), _full((3, 126, 254)),
        _full((100, 50)), _full((1, 50)), _full((2, 63, 126)),
        _full((3, 50, 1)), _full((1, 1)), _full((31, 61)),
        _full((1, 31)), _full((1, 1)),
        _full((3, _NPG, 100)), _full((1, 100)), _full((3, 62, 126)),
        _full((100, 50)), _full((1, 50)), _full((2, 31, 62)),
        _full((3, 50, 1)), _full((1, 1)), _full((15, 29)),
        _full((1, 15)), _full((1, 1)),
    ],
    out_specs=pl.BlockSpec((1, 1, 128), lambda b: (b, 0, 0)),
    out_shape=jax.ShapeDtypeStruct((_B, 1, 128), jnp.float32),
)


_S1W = _pool_mats(254, 3, 2)
_S2W = _pool_mats(126, 2, 2)
_UW = _pool_mats(61, 1, 2)[0]
_S1N = _pool_mats(126, 3, 2)
_S2N = _pool_mats(62, 2, 2)
_UN = _pool_mats(29, 1, 2)[0]


def kernel(x, edge_index, ggnn_w, gru_w_ih, gru_w_hh, gru_b_ih, gru_b_hh,
           cw1_w, cw1_b, cw2_w, cw2_b, cw3_w, cw3_b,
           cn1_w, cn1_b, cn2_w, cn2_b, cn3_w, cn3_b,
           mw_w, mw_b, mn_w, mn_b):
    src, dst = edge_index[0], edge_index[1]
    pad = _EPAD - _E
    srcp = jnp.concatenate([src, jnp.zeros((pad,), jnp.int32)])
    dstp = jnp.concatenate([dst, jnp.full((pad,), _N, jnp.int32)])
    srcp = srcp.reshape(_NTILES, _NCH, _CH)
    dstp = dstp.reshape(_NTILES, _NCH, _CH)
    zeros_acc = jnp.zeros((_NPAD, _H), jnp.float32)

    wih_t = gru_w_ih.T
    whh_t = gru_w_hh.T
    bih = gru_b_ih.reshape(1, 3 * _H)
    bhh = gru_b_hh.reshape(1, 3 * _H)

    h = x
    m = _m0(x, ggnn_w[0])
    for i in range(_L):
        parts = _get_sc_scatter()(m, zeros_acc, srcp, dstp)
        w_next = ggnn_w[(i + 1) % _L]
        h, m = _gru(h, parts, wih_t, whh_t, bih, bhh, w_next)

    xc = jnp.concatenate([h, x], axis=-1)
    zw = xc.reshape(_B, _NPG, _H + _DIN).transpose(0, 2, 1)
    zn = h.reshape(_B, _NPG, _H).transpose(0, 2, 1)

    w1w = cw1_w.transpose(2, 1, 0)
    w2w = cw2_w[:, :, 0].T
    w3w = cw3_w.transpose(2, 1, 0)
    w1n = cn1_w.transpose(2, 1, 0)
    w2n = cn2_w[:, :, 0].T
    w3n = cn3_w.transpose(2, 1, 0)

    out = _head(
        zw, zn,
        w1w, cw1_b.reshape(1, 100), jnp.asarray(_S1W),
        w2w, cw2_b.reshape(1, 50), jnp.asarray(_S2W),
        w3w, cw3_b.reshape(1, 1), jnp.asarray(_UW),
        mw_w, mw_b.reshape(1, 1),
        w1n, cn1_b.reshape(1, 100), jnp.asarray(_S1N),
        w2n, cn2_b.reshape(1, 50), jnp.asarray(_S2N),
        w3n, cn3_b.reshape(1, 1), jnp.asarray(_UN),
        mn_w, mn_b.reshape(1, 1),
    )
    return out[:, 0, :1]
